# baseline (speedup 1.0000x reference)
import functools

import jax
import jax.numpy as jnp
from jax import lax
from jax.experimental import pallas as pl
from jax.experimental.pallas import tpu as pltpu
from jax.experimental.pallas import tpu_sc as plsc

NC = 2
NS = 16
NW = NC * NS
L = 16

XCOLS = 304

XIW = 66
XFW = 30

OFF_WR1 = 0
OFF_BR1 = 16
OFF_WN = 32
OFF_BN = 48
OFF_WB = 64
OFF_WD = 66
OFF_WD2 = 76
OFF_WR2 = 296
OFF_WN2 = 396
WS_LEN = 800


@functools.lru_cache(maxsize=4)
def _make_sc_feats(Bn: int):
  RPT = Bn // NW
  CHUNK = 128
  NCHUNK = RPT // CHUNK
  GP = CHUNK // L

  mesh = plsc.VectorSubcoreMesh(core_axis_name="c", subcore_axis_name="s")

  @functools.partial(
      pl.kernel,
      out_type=jax.ShapeDtypeStruct((XCOLS, Bn), jnp.float32),
      mesh=mesh,
      scratch_types=[
          pltpu.VMEM((RPT * XIW,), jnp.int32),
          pltpu.VMEM((RPT * XFW,), jnp.float32),
          pltpu.VMEM((WS_LEN,), jnp.float32),
          pltpu.VMEM((30000,), jnp.float32),
          pltpu.VMEM((XCOLS, CHUNK), jnp.float32),
          pltpu.SemaphoreType.DMA,
      ],
      compiler_params=pltpu.CompilerParams(needs_layout_passes=False),
  )
  def sc_feats(xi_h, xf_h, ws_h, wid_h, x_h,
               xi_v, xf_v, ws_v, wid_v, out_v, sem):
    wid = lax.axis_index("s") * NC + lax.axis_index("c")
    base = wid * RPT

    c1 = pltpu.async_copy(xi_h.at[pl.ds(base * XIW, RPT * XIW)], xi_v, sem)
    c2 = pltpu.async_copy(xf_h.at[pl.ds(base * XFW, RPT * XFW)], xf_v, sem)
    c3 = pltpu.async_copy(ws_h, ws_v, sem)
    c4 = pltpu.async_copy(wid_h, wid_v, sem)
    for c in (c1, c2, c3, c4):
      c.wait()

    iota = lax.iota(jnp.int32, L)
    zeros = jnp.zeros((L,), jnp.float32)

    wr1_a = ws_v[pl.ds(OFF_WR1, L)]
    br1_a = ws_v[pl.ds(OFF_BR1, L)]
    wn_a = ws_v[pl.ds(OFF_WN, L)]
    bn_a = ws_v[pl.ds(OFF_BN, L)]
    wb_a = ws_v[pl.ds(OFF_WB, L)]
    wr1b = [wr1_a[q] for q in range(5)]
    br1b = [br1_a[q] for q in range(5)]
    wnb = [wn_a[q] for q in range(10)]
    bnb = [bn_a[q] for q in range(10)]
    wb0, wb1d = wb_a[0], wb_a[1] - wb_a[0]

    def do_chunk(c):
      def st(p, lr0, vec):
        out_v[p, pl.ds(lr0, L)] = vec

      @plsc.parallel_loop(0, GP, 1, unroll=2)
      def group_body(gg):
        lr0 = gg * L
        ri = c * CHUNK + lr0 + iota
        rb_i = ri * XIW
        rb_f = ri * XFW

        for p in range(10):
          b = plsc.load_gather(xf_v, [rb_f + p])
          st(p, lr0, wb0 + b * wb1d)

        for p in range(10):
          nv = plsc.load_gather(xf_v, [rb_f + 10 + p])
          n2 = plsc.load_gather(xi_v, [rb_i + p]) * 10 + OFF_WN2
          for q in range(10):
            y = nv * wnb[q] + bnb[q]
            y = jnp.maximum(y, y * jnp.float32(0.01))
            y = jnp.clip(y, jnp.float32(-1.0), jnp.float32(1.0))
            g = plsc.load_gather(ws_v, [n2 + q])
            st(10 + 10 * p + q, lr0, y + g)

        for p in range(10):
          rv = plsc.load_gather(xf_v, [rb_f + 20 + p])
          r2 = plsc.load_gather(xi_v, [rb_i + 10 + p]) * 5 + OFF_WR2
          for q in range(5):
            y = rv * wr1b[q] + br1b[q] + plsc.load_gather(ws_v, [r2 + q])
            st(110 + 5 * p + q, lr0, y)

        for p in range(10):
          d1 = plsc.load_gather(xi_v, [rb_i + 20 + p]) + OFF_WD
          st(160 + p, lr0, plsc.load_gather(ws_v, [d1]))

        for p in range(10):
          d2 = plsc.load_gather(xi_v, [rb_i + 30 + p]) * 5 + OFF_WD2
          for q in range(5):
            st(170 + 5 * p + q, lr0, plsc.load_gather(ws_v, [d2 + q]))

        for p in range(26):
          iv = plsc.load_gather(xi_v, [rb_i + 40 + p]) * 3
          for q in range(3):
            st(220 + 3 * p + q, lr0, plsc.load_gather(wid_v, [iv + q]))

        for p in range(298, XCOLS):
          st(p, lr0, zeros)

      pltpu.sync_copy(out_v, x_h.at[:, pl.ds(base + c * CHUNK, CHUNK)])

    for c in range(NCHUNK):
      do_chunk(c)

  return sc_feats


def _mlp_body(x_ref, w1_ref, b1_ref, w2_ref, b2_ref, w3t_ref, b3_ref, o_ref):
  xb = x_ref[...]
  h = jnp.dot(w1_ref[...], xb, preferred_element_type=jnp.float32)
  h = jnp.maximum(h + b1_ref[...], 0.0)
  h = jnp.dot(w2_ref[...], h, preferred_element_type=jnp.float32)
  h = jnp.maximum(h + b2_ref[...], 0.0)
  o_ref[...] = jnp.dot(h.T, w3t_ref[...],
                       preferred_element_type=jnp.float32) + b3_ref[...]


def _mlp(xT, W1p, b1, W2, b2, W3t, b3r):
  Bn = xT.shape[1]
  BLK = 1024
  return pl.pallas_call(
      _mlp_body,
      grid=(Bn // BLK,),
      in_specs=[
          pl.BlockSpec((XCOLS, BLK), lambda i: (0, i)),
          pl.BlockSpec((128, XCOLS), lambda i: (0, 0)),
          pl.BlockSpec((128, 1), lambda i: (0, 0)),
          pl.BlockSpec((128, 128), lambda i: (0, 0)),
          pl.BlockSpec((128, 1), lambda i: (0, 0)),
          pl.BlockSpec((128, 64), lambda i: (0, 0)),
          pl.BlockSpec((1, 64), lambda i: (0, 0)),
      ],
      out_specs=pl.BlockSpec((BLK, 64), lambda i: (i, 0)),
      out_shape=jax.ShapeDtypeStruct((Bn, 64), jnp.float32),
  )(xT, W1p, b1, W2, b2, W3t, b3r)


def _ws_concat(W_bool, W_dis, W_dis2, W_rat2, W_num2, w_r1, b_r1, w_n, b_n):
  z = lambda n: jnp.zeros((n,), jnp.float32)
  f = lambda a: a.reshape(-1).astype(jnp.float32)
  return jnp.concatenate([
      f(w_r1), z(11), f(b_r1), z(11), f(w_n), z(6), f(b_n), z(6),
      f(W_bool), f(W_dis), f(W_dis2), f(W_rat2), f(W_num2), z(4)])


def kernel(bol, rat, rat2, num, num2, id, dis1, dis2,
           W_bool, W_dis, W_dis2, W_rat2, W_id, W_num2,
           w_r1, b_r1, w_n, b_n, W1, b1, W2, b2, W3, b3):
  Bn = bol.shape[0]
  i32 = jnp.int32
  f32 = jnp.float32
  ws = _ws_concat(W_bool, W_dis, W_dis2, W_rat2, W_num2, w_r1, b_r1, w_n, b_n)
  widf = W_id.reshape(-1).astype(f32)
  W1p = jnp.concatenate([W1, jnp.zeros((128, XCOLS - 298), f32)], axis=1)
  b1r, b2r = b1.reshape(128, 1), b2.reshape(128, 1)
  W3t, b3r = W3.T, b3.reshape(1, 64)

  NSLICE = 2
  Bs = Bn // NSLICE
  sc_feats = _make_sc_feats(Bs)
  outs = []
  for s in range(NSLICE):
    sl = slice(s * Bs, (s + 1) * Bs)
    xi = jnp.concatenate(
        [num2[sl].astype(i32), rat2[sl].astype(i32), dis1[sl].astype(i32),
         dis2[sl].astype(i32), id[sl].astype(i32)], axis=1).reshape(-1)
    xf = jnp.concatenate(
        [bol[sl].astype(f32), num[sl].reshape(Bs, 10).astype(f32),
         rat[sl].reshape(Bs, 10).astype(f32)], axis=1).reshape(-1)
    xT = sc_feats(xi, xf, ws, widf)
    outs.append(_mlp(xT, W1p, b1r, W2, b2r, W3t, b3r))
  return jnp.concatenate(outs, axis=0)


# speedup vs baseline: 1.0480x; 1.0480x over previous
import functools

import jax
import jax.numpy as jnp
from jax import lax
from jax.experimental import pallas as pl
from jax.experimental.pallas import tpu as pltpu
from jax.experimental.pallas import tpu_sc as plsc

NC = 2
NS = 16
NW = NC * NS
L = 16

XCOLS = 304

XIW = 66
XFW = 30

OFF_WR1 = 0
OFF_BR1 = 16
OFF_WN = 32
OFF_BN = 48
OFF_WB = 64
OFF_WD = 66
OFF_WD2 = 76
OFF_WR2 = 296
OFF_WN2 = 396
WS_LEN = 800


@functools.lru_cache(maxsize=4)
def _make_sc_feats(Bn: int):
  RPT = Bn // NW
  CHUNK = 128
  NCHUNK = RPT // CHUNK
  GP = CHUNK // L

  mesh = plsc.VectorSubcoreMesh(core_axis_name="c", subcore_axis_name="s")

  @functools.partial(
      pl.kernel,
      out_type=jax.ShapeDtypeStruct((XCOLS, Bn), jnp.float32),
      mesh=mesh,
      scratch_types=[
          pltpu.VMEM((RPT * XIW,), jnp.int32),
          pltpu.VMEM((RPT * XFW,), jnp.float32),
          pltpu.VMEM((WS_LEN,), jnp.float32),
          pltpu.VMEM((30000,), jnp.float32),
          pltpu.VMEM((XCOLS, CHUNK), jnp.float32),
          pltpu.SemaphoreType.DMA,
      ],
      compiler_params=pltpu.CompilerParams(needs_layout_passes=False),
  )
  def sc_feats(xi_h, xf_h, ws_h, wid_h, x_h,
               xi_v, xf_v, ws_v, wid_v, out_v, sem):
    wid = lax.axis_index("s") * NC + lax.axis_index("c")
    base = wid * RPT

    c1 = pltpu.async_copy(xi_h.at[pl.ds(base * XIW, RPT * XIW)], xi_v, sem)
    c2 = pltpu.async_copy(xf_h.at[pl.ds(base * XFW, RPT * XFW)], xf_v, sem)
    c3 = pltpu.async_copy(ws_h, ws_v, sem)
    c4 = pltpu.async_copy(wid_h, wid_v, sem)
    for c in (c1, c2, c3, c4):
      c.wait()

    iota = lax.iota(jnp.int32, L)
    zeros = jnp.zeros((L,), jnp.float32)

    wr1_a = ws_v[pl.ds(OFF_WR1, L)]
    br1_a = ws_v[pl.ds(OFF_BR1, L)]
    wn_a = ws_v[pl.ds(OFF_WN, L)]
    bn_a = ws_v[pl.ds(OFF_BN, L)]
    wb_a = ws_v[pl.ds(OFF_WB, L)]
    wr1b = [wr1_a[q] for q in range(5)]
    br1b = [br1_a[q] for q in range(5)]
    wnb = [wn_a[q] for q in range(10)]
    bnb = [bn_a[q] for q in range(10)]
    wb0, wb1d = wb_a[0], wb_a[1] - wb_a[0]

    def do_chunk(c, _):
      def st(p, lr0, vec):
        out_v[p, pl.ds(lr0, L)] = vec

      @plsc.parallel_loop(0, GP, 1, unroll=2)
      def group_body(gg):
        lr0 = gg * L
        ri = c * CHUNK + lr0 + iota
        rb_i = ri * XIW
        rb_f = ri * XFW

        for p in range(10):
          b = plsc.load_gather(xf_v, [rb_f + p])
          st(p, lr0, wb0 + b * wb1d)

        for p in range(10):
          nv = plsc.load_gather(xf_v, [rb_f + 10 + p])
          n2 = plsc.load_gather(xi_v, [rb_i + p]) * 10 + OFF_WN2
          for q in range(10):
            y = nv * wnb[q] + bnb[q]
            y = jnp.maximum(y, y * jnp.float32(0.01))
            y = jnp.clip(y, jnp.float32(-1.0), jnp.float32(1.0))
            g = plsc.load_gather(ws_v, [n2 + q])
            st(10 + 10 * p + q, lr0, y + g)

        for p in range(10):
          rv = plsc.load_gather(xf_v, [rb_f + 20 + p])
          r2 = plsc.load_gather(xi_v, [rb_i + 10 + p]) * 5 + OFF_WR2
          for q in range(5):
            y = rv * wr1b[q] + br1b[q] + plsc.load_gather(ws_v, [r2 + q])
            st(110 + 5 * p + q, lr0, y)

        for p in range(10):
          d1 = plsc.load_gather(xi_v, [rb_i + 20 + p]) + OFF_WD
          st(160 + p, lr0, plsc.load_gather(ws_v, [d1]))

        for p in range(10):
          d2 = plsc.load_gather(xi_v, [rb_i + 30 + p]) * 5 + OFF_WD2
          for q in range(5):
            st(170 + 5 * p + q, lr0, plsc.load_gather(ws_v, [d2 + q]))

        for p in range(26):
          iv = plsc.load_gather(xi_v, [rb_i + 40 + p]) * 3
          for q in range(3):
            st(220 + 3 * p + q, lr0, plsc.load_gather(wid_v, [iv + q]))

        for p in range(298, XCOLS):
          st(p, lr0, zeros)

      pltpu.sync_copy(out_v, x_h.at[:, pl.ds(base + c * CHUNK, CHUNK)])
      return 0

    lax.fori_loop(0, NCHUNK, do_chunk, 0)

  return sc_feats


def _mlp_body(x_ref, w1_ref, b1_ref, w2_ref, b2_ref, w3t_ref, b3_ref, o_ref):
  xb = x_ref[...]
  h = jnp.dot(w1_ref[...], xb, preferred_element_type=jnp.float32)
  h = jnp.maximum(h + b1_ref[...], 0.0)
  h = jnp.dot(w2_ref[...], h, preferred_element_type=jnp.float32)
  h = jnp.maximum(h + b2_ref[...], 0.0)
  o_ref[...] = jnp.dot(h.T, w3t_ref[...],
                       preferred_element_type=jnp.float32) + b3_ref[...]


def _mlp(xT, W1p, b1, W2, b2, W3t, b3r):
  Bn = xT.shape[1]
  BLK = 1024
  return pl.pallas_call(
      _mlp_body,
      grid=(Bn // BLK,),
      in_specs=[
          pl.BlockSpec((XCOLS, BLK), lambda i: (0, i)),
          pl.BlockSpec((128, XCOLS), lambda i: (0, 0)),
          pl.BlockSpec((128, 1), lambda i: (0, 0)),
          pl.BlockSpec((128, 128), lambda i: (0, 0)),
          pl.BlockSpec((128, 1), lambda i: (0, 0)),
          pl.BlockSpec((128, 64), lambda i: (0, 0)),
          pl.BlockSpec((1, 64), lambda i: (0, 0)),
      ],
      out_specs=pl.BlockSpec((BLK, 64), lambda i: (i, 0)),
      out_shape=jax.ShapeDtypeStruct((Bn, 64), jnp.float32),
  )(xT, W1p, b1, W2, b2, W3t, b3r)


def _ws_concat(W_bool, W_dis, W_dis2, W_rat2, W_num2, w_r1, b_r1, w_n, b_n):
  z = lambda n: jnp.zeros((n,), jnp.float32)
  f = lambda a: a.reshape(-1).astype(jnp.float32)
  return jnp.concatenate([
      f(w_r1), z(11), f(b_r1), z(11), f(w_n), z(6), f(b_n), z(6),
      f(W_bool), f(W_dis), f(W_dis2), f(W_rat2), f(W_num2), z(4)])


def kernel(bol, rat, rat2, num, num2, id, dis1, dis2,
           W_bool, W_dis, W_dis2, W_rat2, W_id, W_num2,
           w_r1, b_r1, w_n, b_n, W1, b1, W2, b2, W3, b3):
  Bn = bol.shape[0]
  i32 = jnp.int32
  f32 = jnp.float32
  ws = _ws_concat(W_bool, W_dis, W_dis2, W_rat2, W_num2, w_r1, b_r1, w_n, b_n)
  widf = W_id.reshape(-1).astype(f32)
  W1p = jnp.concatenate([W1, jnp.zeros((128, XCOLS - 298), f32)], axis=1)
  b1r, b2r = b1.reshape(128, 1), b2.reshape(128, 1)
  W3t, b3r = W3.T, b3.reshape(1, 64)

  NSLICE = 2
  Bs = Bn // NSLICE
  sc_feats = _make_sc_feats(Bs)
  outs = []
  for s in range(NSLICE):
    sl = slice(s * Bs, (s + 1) * Bs)
    xi = jnp.concatenate(
        [num2[sl].astype(i32), rat2[sl].astype(i32), dis1[sl].astype(i32),
         dis2[sl].astype(i32), id[sl].astype(i32)], axis=1).reshape(-1)
    xf = jnp.concatenate(
        [bol[sl].astype(f32), num[sl].reshape(Bs, 10).astype(f32),
         rat[sl].reshape(Bs, 10).astype(f32)], axis=1).reshape(-1)
    xT = sc_feats(xi, xf, ws, widf)
    outs.append(_mlp(xT, W1p, b1r, W2, b2r, W3t, b3r))
  return jnp.concatenate(outs, axis=0)
